# baseline (speedup 1.0000x reference)
import jax
import jax.numpy as jnp
from jax.experimental import pallas as pl
from jax.experimental.pallas import tpu as pltpu


def _cc_body(cam_ref, w_ref, b_ref, tex_ref, out_ref):
    i = pl.program_id(0)
    ci = cam_ref[i]
    for c in range(3):
        w = w_ref[ci, c]
        b = b_ref[ci, c]
        out_ref[0, c] = tex_ref[0, c] * w + b


_HS = 8


@jax.jit
def kernel(texture, cam, weight, bias):
    B, C, H, W = texture.shape
    dt = texture.dtype
    w_full = jnp.concatenate(
        [jnp.ones((1, C), dt), weight.reshape(-1, C)], axis=0)
    b_full = jnp.concatenate(
        [jnp.zeros((1, C), dt), bias.reshape(-1, C)], axis=0)
    cam32 = cam.astype(jnp.int32)
    hs = H // _HS
    return pl.pallas_call(
        _cc_body,
        grid=(B, _HS),
        in_specs=[
            pl.BlockSpec(memory_space=pltpu.SMEM),
            pl.BlockSpec(memory_space=pltpu.SMEM),
            pl.BlockSpec(memory_space=pltpu.SMEM),
            pl.BlockSpec((1, C, hs, W), lambda i, j: (i, 0, j, 0)),
        ],
        out_specs=pl.BlockSpec((1, C, hs, W), lambda i, j: (i, 0, j, 0)),
        out_shape=jax.ShapeDtypeStruct(texture.shape, dt),
        compiler_params=pltpu.CompilerParams(
            dimension_semantics=("parallel", "parallel")),
    )(cam32, w_full, b_full, texture)


# speedup vs baseline: 1.7405x; 1.7405x over previous
import jax
import jax.numpy as jnp
from jax.experimental import pallas as pl
from jax.experimental.pallas import tpu as pltpu


def _cc_body(cam_ref, w_ref, b_ref, tex_ref, out_ref):
    i = pl.program_id(0)
    c = pl.program_id(1)
    ci = cam_ref[i]
    w = w_ref[ci, c]
    b = b_ref[ci, c]
    out_ref[...] = tex_ref[...] * w + b


@jax.jit
def kernel(texture, cam, weight, bias):
    B, C, H, W = texture.shape
    dt = texture.dtype
    w_full = jnp.concatenate(
        [jnp.ones((1, C), dt), weight.reshape(-1, C)], axis=0)
    b_full = jnp.concatenate(
        [jnp.zeros((1, C), dt), bias.reshape(-1, C)], axis=0)
    cam32 = cam.astype(jnp.int32)
    texf = texture.reshape(B, C * H, W)
    out = pl.pallas_call(
        _cc_body,
        grid=(B, C),
        in_specs=[
            pl.BlockSpec(memory_space=pltpu.SMEM),
            pl.BlockSpec(memory_space=pltpu.SMEM),
            pl.BlockSpec(memory_space=pltpu.SMEM),
            pl.BlockSpec((1, H, W), lambda i, c: (i, c, 0)),
        ],
        out_specs=pl.BlockSpec((1, H, W), lambda i, c: (i, c, 0)),
        out_shape=jax.ShapeDtypeStruct((B, C * H, W), dt),
    )(cam32, w_full, b_full, texf)
    return out.reshape(B, C, H, W)


# speedup vs baseline: 2.5959x; 1.4915x over previous
import jax
import jax.numpy as jnp
from jax.experimental import pallas as pl
from jax.experimental.pallas import tpu as pltpu


def _cc_body(cam_ref, w_ref, b_ref, tex_ref, out_ref):
    i = pl.program_id(0)
    ci = cam_ref[i]
    for c in range(3):
        w = w_ref[ci, c]
        b = b_ref[ci, c]
        out_ref[0, c] = tex_ref[0, c] * w + b


@jax.jit
def kernel(texture, cam, weight, bias):
    B, C, H, W = texture.shape
    dt = texture.dtype
    w_full = jnp.concatenate(
        [jnp.ones((1, C), dt), weight.reshape(-1, C)], axis=0)
    b_full = jnp.concatenate(
        [jnp.zeros((1, C), dt), bias.reshape(-1, C)], axis=0)
    cam32 = cam.astype(jnp.int32)
    return pl.pallas_call(
        _cc_body,
        grid=(B,),
        in_specs=[
            pl.BlockSpec(memory_space=pltpu.SMEM),
            pl.BlockSpec(memory_space=pltpu.SMEM),
            pl.BlockSpec(memory_space=pltpu.SMEM),
            pl.BlockSpec((1, C, H, W), lambda i: (i, 0, 0, 0)),
        ],
        out_specs=pl.BlockSpec((1, C, H, W), lambda i: (i, 0, 0, 0)),
        out_shape=jax.ShapeDtypeStruct(texture.shape, dt),
        compiler_params=pltpu.CompilerParams(
            dimension_semantics=("parallel",)),
    )(cam32, w_full, b_full, texture)
